# Initial kernel scaffold; baseline (speedup 1.0000x reference)
import jax
import jax.numpy as jnp
from jax.experimental import pallas as pl


def kernel(novel_items, novel_userids, item_emb, seg2_out):
    raise NotImplementedError("write your pallas kernel here")


# speedup vs baseline: 3.4493x; 3.4493x over previous
import functools

import jax
import jax.numpy as jnp
from jax import lax
from jax.experimental import pallas as pl
from jax.experimental.pallas import tpu as pltpu
from jax.experimental.pallas import tpu_sc as plsc

T = 32768
V = 100000
D = 64
U = 16

_NC = 2
_NS = 16
_NW = _NC * _NS
_ROWS_PER_W = T // _NW
_CHUNK = 128
_NCHUNK = _ROWS_PER_W // _CHUNK


def _gather_body(table_hbm, idx_hbm, out_hbm, idx_v, rows_v, sem):
    wid = lax.axis_index("s") * _NC + lax.axis_index("c")
    pltpu.sync_copy(idx_hbm.at[wid], idx_v)
    copies = [
        pltpu.async_copy(table_hbm.at[idx_v.at[j]], rows_v.at[j], sem)
        for j in range(_NCHUNK)
    ]
    for c in copies:
        c.wait()
    pltpu.sync_copy(rows_v, out_hbm.at[wid])


def _sc_gather(item_emb, idx):
    mesh = plsc.VectorSubcoreMesh(core_axis_name="c", subcore_axis_name="s")
    k = functools.partial(
        pl.kernel,
        mesh=mesh,
        out_type=jax.ShapeDtypeStruct((_NW, _NCHUNK, _CHUNK, D), jnp.float32),
        scratch_types=[
            pltpu.VMEM((_NCHUNK, _CHUNK), jnp.int32),
            pltpu.VMEM((_NCHUNK, _CHUNK, D), jnp.float32),
            pltpu.SemaphoreType.DMA,
        ],
        compiler_params=pltpu.CompilerParams(use_tc_tiling_on_sc=False),
    )(_gather_body)
    return k(item_emb, idx)


_BT = 512
_GRID = T // _BT


def _morph_body(uid_ref, emb_ref, m_ref, out_ref):
    e = emb_ref[...]
    p = lax.dot_general(e, m_ref[...], (((1,), (0,)), ((), ())),
                        preferred_element_type=jnp.float32)
    uid = uid_ref[0]
    acc = e
    for u in range(U):
        m = (uid == u).astype(jnp.float32)
        acc = acc + p[:, u * D:(u + 1) * D] * m
    n = jnp.sqrt(jnp.sum(acc * acc, axis=1, keepdims=True))
    out_ref[...] = acc / jnp.maximum(n, 1e-12)


def _tc_morph(uid3d, emb2d, m_cat, interpret=False):
    return pl.pallas_call(
        _morph_body,
        grid=(_GRID,),
        in_specs=[
            pl.BlockSpec((1, _BT, 1), lambda i: (i, 0, 0)),
            pl.BlockSpec((_BT, D), lambda i: (i, 0)),
            pl.BlockSpec((D, U * D), lambda i: (0, 0)),
        ],
        out_specs=pl.BlockSpec((_BT, D), lambda i: (i, 0)),
        out_shape=jax.ShapeDtypeStruct((T, D), jnp.float32),
        interpret=interpret,
    )(uid3d, emb2d, m_cat)


def kernel(novel_items, novel_userids, item_emb, seg2_out):
    idx = novel_items.astype(jnp.int32).reshape(_NW, _NCHUNK, _CHUNK)
    emb = _sc_gather(item_emb, idx).reshape(T, D)
    m_cat = seg2_out.transpose(1, 0, 2).reshape(D, U * D)
    uid3d = novel_userids.astype(jnp.int32).reshape(_GRID, _BT, 1)
    return _tc_morph(uid3d, emb, m_cat)


# speedup vs baseline: 4.3182x; 1.2519x over previous
import functools

import jax
import jax.numpy as jnp
from jax import lax
from jax.experimental import pallas as pl
from jax.experimental.pallas import tpu as pltpu
from jax.experimental.pallas import tpu_sc as plsc

T = 32768
V = 100000
D = 64
U = 16

_NC = 2
_NS = 16
_NW = _NC * _NS
_ROWS_PER_W = T // _NW
_CHUNK = 128
_NCHUNK = _ROWS_PER_W // _CHUNK


def _gather_body(table_hbm, idx_hbm, out_hbm, idx_v, rows_v, sem):
    wid = lax.axis_index("s") * _NC + lax.axis_index("c")
    pltpu.sync_copy(idx_hbm.at[wid], idx_v)
    copies = [
        pltpu.async_copy(table_hbm.at[idx_v.at[j]], rows_v.at[j], sem)
        for j in range(_NCHUNK)
    ]
    for c in copies:
        c.wait()
    pltpu.sync_copy(rows_v, out_hbm.at[wid])


def _sc_gather(item_emb, idx):
    mesh = plsc.VectorSubcoreMesh(core_axis_name="c", subcore_axis_name="s")
    k = functools.partial(
        pl.kernel,
        mesh=mesh,
        out_type=jax.ShapeDtypeStruct((_NW, _NCHUNK, _CHUNK, D), jnp.float32),
        scratch_types=[
            pltpu.VMEM((_NCHUNK, _CHUNK), jnp.int32),
            pltpu.VMEM((_NCHUNK, _CHUNK, D), jnp.float32),
            pltpu.SemaphoreType.DMA,
        ],
        compiler_params=pltpu.CompilerParams(use_tc_tiling_on_sc=False),
    )(_gather_body)
    return k(item_emb, idx)


_BT = 512
_GRID = T // _BT


def _morph_body(uid_ref, emb_ref, m_ref, out_ref):
    e = emb_ref[...]
    uid = uid_ref[0]
    lane_u = lax.broadcasted_iota(jnp.int32, (_BT, U * D), 1) // D
    e_rep = jnp.concatenate([e] * U, axis=1)
    b = jnp.where(lane_u == uid, e_rep, 0.0)
    morph = lax.dot_general(b, m_ref[...], (((1,), (0,)), ((), ())),
                            preferred_element_type=jnp.float32)
    acc = e + morph
    n = jnp.sqrt(jnp.sum(acc * acc, axis=1, keepdims=True))
    out_ref[...] = acc / jnp.maximum(n, 1e-12)


def _tc_morph(uid3d, emb2d, m_cat, interpret=False):
    return pl.pallas_call(
        _morph_body,
        grid=(_GRID,),
        in_specs=[
            pl.BlockSpec((1, _BT, 1), lambda i: (i, 0, 0)),
            pl.BlockSpec((_BT, D), lambda i: (i, 0)),
            pl.BlockSpec((U * D, D), lambda i: (0, 0)),
        ],
        out_specs=pl.BlockSpec((_BT, D), lambda i: (i, 0)),
        out_shape=jax.ShapeDtypeStruct((T, D), jnp.float32),
        interpret=interpret,
    )(uid3d, emb2d, m_cat)


def kernel(novel_items, novel_userids, item_emb, seg2_out):
    idx = novel_items.astype(jnp.int32).reshape(_NW, _NCHUNK, _CHUNK)
    emb = _sc_gather(item_emb, idx).reshape(T, D)
    m_flat = seg2_out.reshape(U * D, D)
    uid3d = novel_userids.astype(jnp.int32).reshape(_GRID, _BT, 1)
    return _tc_morph(uid3d, emb, m_flat)


# speedup vs baseline: 4.6221x; 1.0704x over previous
import functools

import jax
import jax.numpy as jnp
from jax import lax
from jax.experimental import pallas as pl
from jax.experimental.pallas import tpu as pltpu
from jax.experimental.pallas import tpu_sc as plsc

T = 32768
V = 100000
D = 64
U = 16
DP = 128

_NC = 2
_NS = 16
_NW = _NC * _NS
_ROWS_PER_W = T // _NW
_CHUNK = 128
_NCHUNK = _ROWS_PER_W // _CHUNK


def _gather_body(table_hbm, idx_hbm, out_hbm, idx_v, rows_v, sem):
    wid = lax.axis_index("s") * _NC + lax.axis_index("c")
    pltpu.sync_copy(idx_hbm.at[wid], idx_v)
    base = wid * _ROWS_PER_W
    for j in range(_NCHUNK):
        pltpu.async_copy(table_hbm.at[idx_v.at[j]], rows_v, sem).wait()
        pltpu.sync_copy(rows_v, out_hbm.at[pl.ds(base + j * _CHUNK, _CHUNK)])


def _sc_gather(table, idx):
    mesh = plsc.VectorSubcoreMesh(core_axis_name="c", subcore_axis_name="s")
    k = functools.partial(
        pl.kernel,
        mesh=mesh,
        out_type=jax.ShapeDtypeStruct((T, DP), jnp.float32),
        scratch_types=[
            pltpu.VMEM((_NCHUNK, _CHUNK), jnp.int32),
            pltpu.VMEM((_CHUNK, DP), jnp.float32),
            pltpu.SemaphoreType.DMA,
        ],
        compiler_params=pltpu.CompilerParams(use_tc_tiling_on_sc=True),
    )(_gather_body)
    return k(table, idx)


_BT = 512
_GRID = T // _BT


def _morph_body(bnd_ref, emb_ref, m_ref, out_ref):
    g = pl.program_id(0)
    v = emb_ref[...]
    e = v[:, :D]
    tok = g * _BT + lax.broadcasted_iota(jnp.int32, (_BT, 1), 0)
    parts = []
    for u in range(U):
        m_u = jnp.logical_and(tok >= bnd_ref[u], tok < bnd_ref[u + 1])
        parts.append(e * m_u.astype(jnp.float32))
    b = jnp.concatenate(parts, axis=1)
    morph = lax.dot_general(b, m_ref[...], (((1,), (0,)), ((), ())),
                            preferred_element_type=jnp.float32)
    acc = e + morph
    n = jnp.sqrt(jnp.sum(acc * acc, axis=1, keepdims=True))
    out_ref[...] = acc / jnp.maximum(n, 1e-12)


def _tc_morph(bnd, emb128, m_flat, interpret=False):
    grid_spec = pltpu.PrefetchScalarGridSpec(
        num_scalar_prefetch=1,
        grid=(_GRID,),
        in_specs=[
            pl.BlockSpec((_BT, DP), lambda i, bnd: (i, 0)),
            pl.BlockSpec((U * D, D), lambda i, bnd: (0, 0)),
        ],
        out_specs=pl.BlockSpec((_BT, D), lambda i, bnd: (i, 0)),
    )
    return pl.pallas_call(
        _morph_body,
        grid_spec=grid_spec,
        out_shape=jax.ShapeDtypeStruct((T, D), jnp.float32),
        interpret=interpret,
    )(bnd, emb128, m_flat)


def kernel(novel_items, novel_userids, item_emb, seg2_out):
    idx = novel_items.astype(jnp.int32).reshape(_NW, _NCHUNK, _CHUNK)
    tbl = jnp.concatenate(
        [item_emb, jnp.zeros((V, DP - D), jnp.float32)], axis=1)
    emb128 = _sc_gather(tbl, idx)
    m_flat = seg2_out.reshape(U * D, D)
    bnd = jnp.searchsorted(
        novel_userids, jnp.arange(U + 1, dtype=novel_userids.dtype)
    ).astype(jnp.int32)
    return _tc_morph(bnd, emb128, m_flat)


# speedup vs baseline: 5.1289x; 1.1096x over previous
import functools

import jax
import jax.numpy as jnp
from jax import lax
from jax.experimental import pallas as pl
from jax.experimental.pallas import tpu as pltpu
from jax.experimental.pallas import tpu_sc as plsc

T = 32768
V = 100000
D = 64
U = 16
DP = 128

_NC = 2
_NS = 16
_NW = _NC * _NS
_ROWS_PER_W = T // _NW
_CHUNK = 128
_NCHUNK = _ROWS_PER_W // _CHUNK


def _gather_body(table_hbm, idx_hbm, out_hbm, idx_v, rows_v, sem):
    wid = lax.axis_index("s") * _NC + lax.axis_index("c")
    pltpu.sync_copy(idx_hbm.at[wid], idx_v)
    base = wid * _ROWS_PER_W
    for j in range(_NCHUNK):
        pltpu.async_copy(table_hbm.at[idx_v.at[j]], rows_v, sem).wait()
        pltpu.sync_copy(rows_v, out_hbm.at[pl.ds(base + j * _CHUNK, _CHUNK)])


def _sc_gather(table, idx):
    mesh = plsc.VectorSubcoreMesh(core_axis_name="c", subcore_axis_name="s")
    k = functools.partial(
        pl.kernel,
        mesh=mesh,
        out_type=jax.ShapeDtypeStruct((T, DP), jnp.float32),
        scratch_types=[
            pltpu.VMEM((_NCHUNK, _CHUNK), jnp.int32),
            pltpu.VMEM((_CHUNK, DP), jnp.float32),
            pltpu.SemaphoreType.DMA,
        ],
        compiler_params=pltpu.CompilerParams(use_tc_tiling_on_sc=True),
    )(_gather_body)
    return k(table, idx)


_BT = 1024
_GRID = T // _BT


def _morph_body(bnd_ref, emb_ref, m_ref, out_ref):
    g = pl.program_id(0)
    v = emb_ref[...]
    e = v[:, :D]
    e_bf = e.astype(jnp.bfloat16)
    tok = g * _BT + lax.broadcasted_iota(jnp.int32, (_BT, 1), 0)
    parts = []
    for u in range(U):
        m_u = jnp.logical_and(tok >= bnd_ref[u], tok < bnd_ref[u + 1])
        parts.append(e_bf * m_u.astype(jnp.bfloat16))
    b = jnp.concatenate(parts, axis=1)
    morph = lax.dot_general(b, m_ref[...], (((1,), (0,)), ((), ())),
                            preferred_element_type=jnp.float32)
    acc = e + morph
    n = jnp.sqrt(jnp.sum(acc * acc, axis=1, keepdims=True))
    out_ref[...] = acc / jnp.maximum(n, 1e-12)


def _tc_morph(bnd, emb128, m_flat, interpret=False):
    grid_spec = pltpu.PrefetchScalarGridSpec(
        num_scalar_prefetch=1,
        grid=(_GRID,),
        in_specs=[
            pl.BlockSpec((_BT, DP), lambda i, bnd: (i, 0)),
            pl.BlockSpec((U * D, D), lambda i, bnd: (0, 0)),
        ],
        out_specs=pl.BlockSpec((_BT, D), lambda i, bnd: (i, 0)),
    )
    return pl.pallas_call(
        _morph_body,
        grid_spec=grid_spec,
        out_shape=jax.ShapeDtypeStruct((T, D), jnp.float32),
        interpret=interpret,
    )(bnd, emb128, m_flat)


def kernel(novel_items, novel_userids, item_emb, seg2_out):
    idx = novel_items.astype(jnp.int32).reshape(_NW, _NCHUNK, _CHUNK)
    tbl = jnp.concatenate(
        [item_emb, jnp.zeros((V, DP - D), jnp.float32)], axis=1)
    emb128 = _sc_gather(tbl, idx)
    m_flat = seg2_out.reshape(U * D, D).astype(jnp.bfloat16)
    bnd = jnp.searchsorted(
        novel_userids, jnp.arange(U + 1, dtype=novel_userids.dtype)
    ).astype(jnp.int32)
    return _tc_morph(bnd, emb128, m_flat)
